# baseline (speedup 1.0000x reference)
import functools

import jax
import jax.numpy as jnp
from jax import lax
from jax.experimental import pallas as pl
from jax.experimental.pallas import tpu as pltpu
from jax.experimental.pallas import tpu_sc as plsc

_N = 4096
_TP = 256
_KA = 8
_QR = 256
_NTC = _N - _QR
_NW = 32
_RW = _QR // (_NW // 4)


def _norm_split(sq):
    hi = jnp.floor(sq * (1.0 / 256.0)) * 256.0
    return hi, sq - hi


def _vox_t(ct):
    cn = jnp.where(jnp.isnan(ct), jnp.inf, ct)
    mn = jnp.min(cn, axis=1, keepdims=True)
    return ((ct - mn) / 0.1).astype(jnp.int32).astype(jnp.float32)


def _tc_body(p_ref, g_ref, o_ref, oc_ref, pa_ref, ga_ref):
    p = p_ref[0]
    g = g_ref[0]
    pv = _vox_t(p)
    gv = _vox_t(g)
    shift = jnp.floor(jnp.maximum(jnp.max(pv, axis=1, keepdims=True),
                                  jnp.max(gv, axis=1, keepdims=True)) * 0.5)
    pv = pv - shift
    gv = gv - shift

    one_row = jnp.ones((1, _N), jnp.float32)
    pa_ref[7:8, :] = jnp.zeros((1, _N), jnp.float32)
    ga_ref[7:8, :] = jnp.zeros((1, _N), jnp.float32)
    total = jnp.float32(0.0)
    for mode, (pt, gt) in enumerate(((p, g), (pv, gv))):
        xxh, xxl = _norm_split(jnp.sum(pt * pt, axis=0, keepdims=True))
        yyh, yyl = _norm_split(jnp.sum(gt * gt, axis=0, keepdims=True))
        pa_ref[0:3, :] = pt
        pa_ref[3:4, :] = xxh
        pa_ref[4:5, :] = xxl
        pa_ref[5:6, :] = one_row
        pa_ref[6:7, :] = one_row
        ga_ref[0:3, :] = -2.0 * gt
        ga_ref[3:4, :] = one_row
        ga_ref[4:5, :] = one_row
        ga_ref[5:6, :] = yyh
        ga_ref[6:7, :] = yyl
        ga = ga_ref[...]

        def step(c, carry):
            cacc, s1 = carry
            pc = pa_ref[:, pl.ds(c * _TP, _TP)]
            d = jax.lax.dot_general(
                pc, ga, (((0,), (0,)), ((), ())),
                preferred_element_type=jnp.float32)
            s1 = s1 + jnp.sum(jnp.min(d, axis=1))
            cacc = jnp.minimum(cacc, jnp.min(d, axis=0, keepdims=True))
            return cacc, s1

        nchunk = (_NTC if mode == 0 else _N) // _TP
        cacc0 = jnp.full((1, _N), jnp.inf, dtype=jnp.float32)
        cacc, s1 = jax.lax.fori_loop(
            0, nchunk, step, (cacc0, jnp.float32(0.0)), unroll=nchunk)
        if mode == 0:
            oc_ref[0, :, :] = jnp.broadcast_to(cacc, (8, _N))
            total = total + s1
        else:
            total = total + s1 + jnp.sum(cacc)
    b = pl.program_id(0)

    @pl.when(b == 0)
    def _():
        o_ref[0] = jnp.float32(0.0)

    o_ref[0] = o_ref[0] + total * jnp.float32(1.0 / (4 * _N))


def _sc_body(pg_ref, outs_ref, outc_ref, cand_v, q_v, row_v, col_v):
    w = lax.axis_index("s") * 2 + lax.axis_index("c")
    b = w // 8
    sub = w % 8
    pltpu.sync_copy(pg_ref.at[b + 4], cand_v)
    pltpu.sync_copy(pg_ref.at[b], q_v)
    qoff = _NTC + sub * _RW
    inf16 = jnp.full((16,), jnp.inf, dtype=jnp.float32)

    def initstep(jv, acc):
        col_v[pl.ds(jv * 16, 16)] = inf16
        return acc

    lax.fori_loop(0, _N // 16, initstep, jnp.int32(0))

    qv = [[q_v[c, pl.ds(qoff + 16 * k, 16)] for k in range(_RW // 16)]
          for c in range(3)]
    _QB = 4
    rowsum = jnp.zeros((16,), jnp.float32)
    for ib in range(_RW // _QB):
        qs = []
        for t in range(_QB):
            i = ib * _QB + t
            k, l = divmod(i, 16)
            qs.append((qv[0][k][l], qv[1][k][l], qv[2][k][l]))

        def cstep(jv, carry):
            cx = cand_v[0, pl.ds(jv * 16, 16)]
            cy = cand_v[1, pl.ds(jv * 16, 16)]
            cz = cand_v[2, pl.ds(jv * 16, 16)]
            col = col_v[pl.ds(jv * 16, 16)]
            newm = []
            for t in range(_QB):
                dx = cx - qs[t][0]
                dy = cy - qs[t][1]
                dz = cz - qs[t][2]
                d = dx * dx + dy * dy + dz * dz
                newm.append(jnp.minimum(carry[t], d))
                col = jnp.minimum(col, d)
            col_v[pl.ds(jv * 16, 16)] = col
            return tuple(newm)

        m = lax.fori_loop(0, _N // 16, cstep, tuple([inf16] * _QB))
        lanes = lax.iota(jnp.int32, 16)
        dnums = lax.GatherDimensionNumbers(
            offset_dims=(), collapsed_slice_dims=(0,), start_index_map=(0,))
        for t in range(_QB):
            v = m[t]
            for s in (8, 4, 2, 1):
                perm = jnp.bitwise_xor(lanes, s)
                vp = lax.gather(v, perm[:, None], dnums, slice_sizes=(1,),
                                mode=lax.GatherScatterMode.PROMISE_IN_BOUNDS)
                v = jnp.minimum(v, vp)
            rowsum = rowsum + v
    row_v[...] = rowsum
    pltpu.sync_copy(row_v, outs_ref.at[w])
    pltpu.sync_copy(col_v, outc_ref.at[w])


@jax.jit
def kernel(preds, gts):
    pg_t = jnp.concatenate([preds, gts], axis=0).transpose(0, 2, 1)

    sc = pl.kernel(
        _sc_body,
        mesh=plsc.VectorSubcoreMesh(core_axis_name="c", subcore_axis_name="s"),
        out_type=[
            jax.ShapeDtypeStruct((_NW, 16), jnp.float32),
            jax.ShapeDtypeStruct((_NW, _N), jnp.float32),
        ],
        scratch_types=[
            pltpu.VMEM((3, _N), jnp.float32),
            pltpu.VMEM((3, _N), jnp.float32),
            pltpu.VMEM((16,), jnp.float32),
            pltpu.VMEM((_N,), jnp.float32),
        ],
    )
    sc_s, sc_c = sc(pg_t)

    tc_scalar, tc_col = pl.pallas_call(
        _tc_body,
        grid=(4,),
        in_specs=[
            pl.BlockSpec((1, 3, _N), lambda b: (b, 0, 0)),
            pl.BlockSpec((1, 3, _N), lambda b: (b + 4, 0, 0)),
        ],
        out_specs=[
            pl.BlockSpec(memory_space=pltpu.SMEM),
            pl.BlockSpec((1, 8, _N), lambda b: (b, 0, 0)),
        ],
        out_shape=[
            jax.ShapeDtypeStruct((1,), jnp.float32),
            jax.ShapeDtypeStruct((4, 8, _N), jnp.float32),
        ],
        scratch_shapes=[
            pltpu.VMEM((_KA, _N), jnp.float32),
            pltpu.VMEM((_KA, _N), jnp.float32),
        ],
    )(pg_t, pg_t)

    col = jnp.minimum(jnp.min(sc_c.reshape(4, 8, _N), axis=1),
                      tc_col[:, 0, :])
    rest = jnp.sum(sc_s[:, 0]) + jnp.sum(col)
    return tc_scalar[0] + rest / jnp.float32(4 * _N)


# speedup vs baseline: 1.2607x; 1.2607x over previous
import functools

import jax
import jax.numpy as jnp
from jax.experimental import pallas as pl
from jax.experimental.pallas import tpu as pltpu

_N = 4096
_TP = 256
_KA = 8


def _norm_split(sq):
    hi = jnp.floor(sq * (1.0 / 256.0)) * 256.0
    return hi, sq - hi


def _vox_t(ct):
    cn = jnp.where(jnp.isnan(ct), jnp.inf, ct)
    mn = jnp.min(cn, axis=1, keepdims=True)
    return ((ct - mn) / 0.1).astype(jnp.int32).astype(jnp.float32)


def _chamfer_body(p_ref, g_ref, o_ref, pa_ref, ga_ref):
    p = p_ref[0]
    g = g_ref[0]
    pv = _vox_t(p)
    gv = _vox_t(g)
    shift = jnp.floor(jnp.maximum(jnp.max(pv, axis=1, keepdims=True),
                                  jnp.max(gv, axis=1, keepdims=True)) * 0.5)
    pv = pv - shift
    gv = gv - shift

    one_row = jnp.ones((1, _N), jnp.float32)
    pa_ref[7:8, :] = jnp.zeros((1, _N), jnp.float32)
    ga_ref[7:8, :] = jnp.zeros((1, _N), jnp.float32)
    total = jnp.float32(0.0)
    for pt, gt in ((p, g), (pv, gv)):
        xxh, xxl = _norm_split(jnp.sum(pt * pt, axis=0, keepdims=True))
        yyh, yyl = _norm_split(jnp.sum(gt * gt, axis=0, keepdims=True))
        pa_ref[0:3, :] = pt
        pa_ref[3:4, :] = xxh
        pa_ref[4:5, :] = xxl
        pa_ref[5:6, :] = one_row
        pa_ref[6:7, :] = one_row
        ga_ref[0:3, :] = -2.0 * gt
        ga_ref[3:4, :] = one_row
        ga_ref[4:5, :] = one_row
        ga_ref[5:6, :] = yyh
        ga_ref[6:7, :] = yyl
        ga = ga_ref[...]

        def step(c, carry):
            cacc, s1 = carry
            pc = pa_ref[:, pl.ds(c * _TP, _TP)]
            d = jax.lax.dot_general(
                pc, ga, (((0,), (0,)), ((), ())),
                preferred_element_type=jnp.float32)
            s1 = s1 + jnp.sum(jnp.min(d, axis=1))
            cacc = jnp.minimum(cacc, jnp.min(d, axis=0, keepdims=True))
            return cacc, s1

        cacc0 = jnp.full((1, _N), jnp.inf, dtype=jnp.float32)
        cacc, s1 = jax.lax.fori_loop(
            0, _N // _TP, step, (cacc0, jnp.float32(0.0)), unroll=16)
        total = total + s1 + jnp.sum(cacc)
    b = pl.program_id(0)

    @pl.when(b == 0)
    def _():
        o_ref[0] = jnp.float32(0.0)

    o_ref[0] = o_ref[0] + total * jnp.float32(1.0 / (4 * _N))


@jax.jit
def kernel(preds, gts):
    p_t = preds.transpose(0, 2, 1)
    g_t = gts.transpose(0, 2, 1)

    sums = pl.pallas_call(
        _chamfer_body,
        grid=(4,),
        in_specs=[
            pl.BlockSpec((1, 3, _N), lambda b: (b, 0, 0)),
            pl.BlockSpec((1, 3, _N), lambda b: (b, 0, 0)),
        ],
        out_specs=pl.BlockSpec(memory_space=pltpu.SMEM),
        out_shape=jax.ShapeDtypeStruct((1,), jnp.float32),
        scratch_shapes=[
            pltpu.VMEM((_KA, _N), jnp.float32),
            pltpu.VMEM((_KA, _N), jnp.float32),
        ],
    )(p_t, g_t)

    return sums[0]
